# Initial kernel scaffold; baseline (speedup 1.0000x reference)
import functools

import jax
import jax.numpy as jnp
from jax import lax
from jax.experimental import pallas as pl
from jax.experimental.pallas import tpu as pltpu
from jax.experimental.pallas import tpu_sc as plsc

NUM_NODES = 50000
BATCH = 10000
NUM_EDGES = 320000
FEAT = 128

NC = 2
NS = 16
NW = NC * NS
E_TILE = NUM_EDGES // NW
K = 80
NCHUNK = E_TILE // K
RPT = BATCH // NS
CW = 16
ZROWS = 125


def _phase1_body(src_hbm, seg_hbm, table_hbm, sums_hbm, cnts_hbm,
                 src_v, seg_v, rows_v, ones_v, zrow_v, zcnt_v,
                 acc_sh, cnt_sh, sem):
  c = lax.axis_index("c")
  s = lax.axis_index("s")
  wid = c * NS + s

  zero16 = jnp.zeros((16,), jnp.float32)
  one16 = jnp.ones((16,), jnp.float32)

  def init_zrow(i, carry):
    for q in range(8):
      zrow_v[i, pl.ds(q * 16, 16)] = zero16
    return carry

  lax.fori_loop(0, ZROWS, init_zrow, 0)

  def init_zcnt(i, carry):
    zcnt_v[i, :] = zero16
    return carry

  lax.fori_loop(0, RPT, init_zcnt, 0)

  def init_ones(i, carry):
    ones_v[i, :] = one16
    return carry

  lax.fori_loop(0, K, init_ones, 0)

  row0 = s * RPT
  for r in range(RPT // ZROWS):
    pltpu.sync_copy(zrow_v, acc_sh.at[pl.ds(row0 + r * ZROWS, ZROWS)])
  pltpu.sync_copy(zcnt_v, cnt_sh.at[pl.ds(row0, RPT)])
  plsc.subcore_barrier()

  pltpu.sync_copy(src_hbm.at[pl.ds(wid * E_TILE, E_TILE)], src_v)
  pltpu.sync_copy(seg_hbm.at[pl.ds(wid * NCHUNK, NCHUNK)], seg_v)

  def chunk(j, carry):
    idx = src_v.at[pl.ds(j * K, K)]
    pltpu.async_copy(table_hbm.at[idx], rows_v, sem).wait()
    segrow = seg_v.at[j]
    pltpu.sync_copy(rows_v, acc_sh.at[segrow], add=True)
    pltpu.sync_copy(ones_v, cnt_sh.at[segrow], add=True)
    return carry

  lax.fori_loop(0, NCHUNK, chunk, 0)
  plsc.subcore_barrier()

  pltpu.sync_copy(acc_sh.at[pl.ds(row0, RPT)], sums_hbm.at[c, pl.ds(row0, RPT)])
  pltpu.sync_copy(cnt_sh.at[pl.ds(row0, RPT)], cnts_hbm.at[c, pl.ds(row0, RPT)])


_phase1 = functools.partial(
    pl.kernel,
    out_type=(
        jax.ShapeDtypeStruct((NC, BATCH, FEAT), jnp.float32),
        jax.ShapeDtypeStruct((NC, BATCH, CW), jnp.float32),
    ),
    mesh=plsc.VectorSubcoreMesh(
        core_axis_name="c", subcore_axis_name="s",
        num_cores=NC, num_subcores=NS),
    scratch_types=[
        pltpu.VMEM((E_TILE,), jnp.int32),
        pltpu.VMEM((NCHUNK, K), jnp.int32),
        pltpu.VMEM((K, FEAT), jnp.float32),
        pltpu.VMEM((K, CW), jnp.float32),
        pltpu.VMEM((ZROWS, FEAT), jnp.float32),
        pltpu.VMEM((RPT, CW), jnp.float32),
        pltpu.VMEM_SHARED((BATCH, FEAT), jnp.float32),
        pltpu.VMEM_SHARED((BATCH, CW), jnp.float32),
        pltpu.SemaphoreType.DMA,
    ],
)(_phase1_body)


RB = 1000


def _phase2_body(a_ref, c_ref, o_ref):
  sums = a_ref[0] + a_ref[1]
  cnt = c_ref[0, :, 0:1] + c_ref[1, :, 0:1]
  o_ref[...] = jnp.maximum(sums / jnp.maximum(cnt, 1.0), 0.0)


def _phase2(sums, cnts):
  return pl.pallas_call(
      _phase2_body,
      grid=(BATCH // RB,),
      in_specs=[
          pl.BlockSpec((NC, RB, FEAT), lambda i: (0, i, 0)),
          pl.BlockSpec((NC, RB, CW), lambda i: (0, i, 0)),
      ],
      out_specs=pl.BlockSpec((RB, FEAT), lambda i: (i, 0)),
      out_shape=jax.ShapeDtypeStruct((BATCH, FEAT), jnp.float32),
  )(sums, cnts)


@jax.jit
def kernel(neigh_src_ids, neigh_seg_ids, features_table):
  src = neigh_src_ids.astype(jnp.int32)
  seg2d = neigh_seg_ids.astype(jnp.int32).reshape(NUM_EDGES // K, K)
  sums, cnts = _phase1(src, seg2d, features_table)
  return _phase2(sums, cnts)


# speedup vs baseline: 5.0160x; 5.0160x over previous
import functools

import jax
import jax.numpy as jnp
from jax import lax
from jax.experimental import pallas as pl
from jax.experimental.pallas import tpu as pltpu
from jax.experimental.pallas import tpu_sc as plsc

NUM_NODES = 50000
BATCH = 10000
NUM_EDGES = 320000
FEAT = 128

NC = 2
NS = 16
HF = FEAT // NC
E_TILE = NUM_EDGES // NS
K = 80
NCHUNK = E_TILE // K
BATCHP = 10240
RPT = BATCHP // NS
CW = 16
ZROWS = 128


def _phase1_body(src_hbm, seg_hbm, tbl_hbm, sums_hbm, cnts_hbm,
                 src_v, seg_v, rows_v, ones_v, zrow_v, zcnt_v,
                 acc_sh, cnt_sh, sem):
  c = lax.axis_index("c")
  s = lax.axis_index("s")

  zero16 = jnp.zeros((16,), jnp.float32)
  one16 = jnp.ones((16,), jnp.float32)

  def init_zrow(i, carry):
    for q in range(HF // 16):
      zrow_v[i, pl.ds(q * 16, 16)] = zero16
    return carry

  lax.fori_loop(0, ZROWS, init_zrow, 0)

  def init_zcnt(i, carry):
    zcnt_v[i, :] = zero16
    return carry

  lax.fori_loop(0, RPT, init_zcnt, 0)

  def init_ones(i, carry):
    ones_v[i, :] = one16
    return carry

  lax.fori_loop(0, K, init_ones, 0)

  row0 = s * RPT
  for r in range(RPT // ZROWS):
    pltpu.sync_copy(zrow_v, acc_sh.at[pl.ds(row0 + r * ZROWS, ZROWS)])
  pltpu.sync_copy(zcnt_v, cnt_sh.at[pl.ds(row0, RPT)])
  plsc.subcore_barrier()

  pltpu.sync_copy(src_hbm.at[pl.ds(s * E_TILE, E_TILE)], src_v)
  pltpu.sync_copy(seg_hbm.at[s], seg_v)

  def chunk(j, carry):
    idx = src_v.at[pl.ds(j * K, K)]
    pltpu.async_copy(tbl_hbm.at[c].at[idx], rows_v, sem).wait()
    segrow = seg_v.at[j]
    pltpu.sync_copy(rows_v, acc_sh.at[segrow], add=True)

    @pl.when((j < NCHUNK // 2) == (c == 0))
    def _():
      pltpu.sync_copy(ones_v, cnt_sh.at[segrow], add=True)

    return carry

  lax.fori_loop(0, NCHUNK, chunk, 0)
  plsc.subcore_barrier()

  pltpu.sync_copy(acc_sh.at[pl.ds(row0, RPT)], sums_hbm.at[c, pl.ds(row0, RPT)])
  pltpu.sync_copy(cnt_sh.at[pl.ds(row0, RPT)], cnts_hbm.at[c, pl.ds(row0, RPT)])


_phase1 = functools.partial(
    pl.kernel,
    out_type=(
        jax.ShapeDtypeStruct((NC, BATCHP, HF), jnp.float32),
        jax.ShapeDtypeStruct((NC, BATCHP, CW), jnp.float32),
    ),
    mesh=plsc.VectorSubcoreMesh(
        core_axis_name="c", subcore_axis_name="s",
        num_cores=NC, num_subcores=NS),
    scratch_types=[
        pltpu.VMEM((E_TILE,), jnp.int32),
        pltpu.VMEM((NCHUNK, K), jnp.int32),
        pltpu.VMEM((K, HF), jnp.float32),
        pltpu.VMEM((K, CW), jnp.float32),
        pltpu.VMEM((ZROWS, HF), jnp.float32),
        pltpu.VMEM((RPT, CW), jnp.float32),
        pltpu.VMEM_SHARED((BATCHP, HF), jnp.float32),
        pltpu.VMEM_SHARED((BATCHP, CW), jnp.float32),
        pltpu.SemaphoreType.DMA,
    ],
    compiler_params=pltpu.CompilerParams(use_tc_tiling_on_sc=False),
)(_phase1_body)


RB = 1000


def _phase2_body(a_ref, c_ref, o_ref):
  cnt = jnp.maximum(c_ref[0, :, 0:1] + c_ref[1, :, 0:1], 1.0)
  lo = jnp.maximum(a_ref[0] / cnt, 0.0)
  hi = jnp.maximum(a_ref[1] / cnt, 0.0)
  o_ref[...] = jnp.concatenate([lo, hi], axis=1)


def _phase2(sums, cnts):
  return pl.pallas_call(
      _phase2_body,
      grid=(BATCH // RB,),
      in_specs=[
          pl.BlockSpec((NC, RB, HF), lambda i: (0, i, 0)),
          pl.BlockSpec((NC, RB, CW), lambda i: (0, i, 0)),
      ],
      out_specs=pl.BlockSpec((RB, FEAT), lambda i: (i, 0)),
      out_shape=jax.ShapeDtypeStruct((BATCH, FEAT), jnp.float32),
  )(sums, cnts)


@jax.jit
def kernel(neigh_src_ids, neigh_seg_ids, features_table):
  src = neigh_src_ids.astype(jnp.int32)
  seg3d = neigh_seg_ids.astype(jnp.int32).reshape(NS, NCHUNK, K)
  tbl = features_table.reshape(NUM_NODES, NC, HF).transpose(1, 0, 2)
  sums, cnts = _phase1(src, seg3d, tbl)
  return _phase2(sums, cnts)


# speedup vs baseline: 12.2411x; 2.4404x over previous
import functools

import jax
import jax.numpy as jnp
from jax import lax
from jax.experimental import pallas as pl
from jax.experimental.pallas import tpu as pltpu
from jax.experimental.pallas import tpu_sc as plsc

NUM_NODES = 50000
BATCH = 10000
NUM_EDGES = 320000
FEAT = 128

NC = 2
NS = 16
HF = FEAT // NC
E_TILE = NUM_EDGES // NS
K = 40
NCHUNK = E_TILE // K
BATCHP = 10240
RPT = BATCHP // NS
CW = 16
ZROWS = 64
NBUF = 5
NGRP = NCHUNK // NBUF


def _phase1_body(src_hbm, seg_hbm, tbl_hbm, sums_hbm, cnts_hbm,
                 src_v, seg_v, rows_v, ones_v, zrow_v, zcnt_v,
                 acc_sh, cnt_sh, sem):
  c = lax.axis_index("c")
  s = lax.axis_index("s")

  zero16 = jnp.zeros((16,), jnp.float32)
  one16 = jnp.ones((16,), jnp.float32)

  def init_zrow(i, carry):
    for q in range(HF // 16):
      zrow_v[i, pl.ds(q * 16, 16)] = zero16
    return carry

  lax.fori_loop(0, ZROWS, init_zrow, 0)

  def init_zcnt(i, carry):
    zcnt_v[i, :] = zero16
    return carry

  lax.fori_loop(0, ZROWS, init_zcnt, 0)

  def init_ones(i, carry):
    ones_v[i, :] = one16
    return carry

  lax.fori_loop(0, K, init_ones, 0)

  row0 = s * RPT
  for r in range(RPT // ZROWS):
    pltpu.sync_copy(zrow_v, acc_sh.at[pl.ds(row0 + r * ZROWS, ZROWS)])
  for r in range(RPT // ZROWS):
    pltpu.sync_copy(zcnt_v, cnt_sh.at[pl.ds(row0 + r * ZROWS, ZROWS)])
  plsc.subcore_barrier()

  pltpu.sync_copy(src_hbm.at[pl.ds(s * E_TILE, E_TILE)], src_v)
  pltpu.sync_copy(seg_hbm.at[s], seg_v)

  def mk_idx(i, carry):
    v = src_v[pl.ds(i * 16, 16)]
    src_v[pl.ds(i * 16, 16)] = v * 2 + c
    return carry

  lax.fori_loop(0, E_TILE // 16, mk_idx, 0)

  for b in range(NBUF):
    idx = src_v.at[pl.ds(b * K, K)]
    pltpu.async_copy(tbl_hbm.at[idx], rows_v.at[b], sem.at[b])

  def group(g, carry):
    for b in range(NBUF):
      j = g * NBUF + b
      idx = src_v.at[pl.ds(j * K, K)]
      pltpu.make_async_copy(tbl_hbm.at[idx], rows_v.at[b],
                            sem.at[b]).wait()
      segrow = seg_v.at[j]
      pltpu.sync_copy(rows_v.at[b], acc_sh.at[segrow], add=True)

      @pl.when((j < NCHUNK // 2) == (c == 0))
      def _():
        pltpu.sync_copy(ones_v, cnt_sh.at[segrow], add=True)

      @pl.when(g < NGRP - 1)
      def _():
        idx2 = src_v.at[pl.ds((j + NBUF) * K, K)]
        pltpu.async_copy(tbl_hbm.at[idx2], rows_v.at[b], sem.at[b])

    return carry

  lax.fori_loop(0, NGRP, group, 0)
  plsc.subcore_barrier()

  pltpu.sync_copy(acc_sh.at[pl.ds(row0, RPT)], sums_hbm.at[c, pl.ds(row0, RPT)])
  pltpu.sync_copy(cnt_sh.at[pl.ds(row0, RPT)], cnts_hbm.at[c, pl.ds(row0, RPT)])


_phase1 = functools.partial(
    pl.kernel,
    out_type=(
        jax.ShapeDtypeStruct((NC, BATCHP, HF), jnp.float32),
        jax.ShapeDtypeStruct((NC, BATCHP, CW), jnp.float32),
    ),
    mesh=plsc.VectorSubcoreMesh(
        core_axis_name="c", subcore_axis_name="s",
        num_cores=NC, num_subcores=NS),
    scratch_types=[
        pltpu.VMEM((E_TILE,), jnp.int32),
        pltpu.VMEM((NCHUNK, K), jnp.int32),
        pltpu.VMEM((NBUF, K, HF), jnp.float32),
        pltpu.VMEM((K, CW), jnp.float32),
        pltpu.VMEM((ZROWS, HF), jnp.float32),
        pltpu.VMEM((ZROWS, CW), jnp.float32),
        pltpu.VMEM_SHARED((BATCHP, HF), jnp.float32),
        pltpu.VMEM_SHARED((BATCHP, CW), jnp.float32),
        pltpu.SemaphoreType.DMA((NBUF,)),
    ],
    compiler_params=pltpu.CompilerParams(use_tc_tiling_on_sc=False),
)(_phase1_body)


RB = 1000


def _phase2_body(a_ref, c_ref, o_ref):
  cnt = jnp.maximum(c_ref[0, :, 0:1] + c_ref[1, :, 0:1], 1.0)
  lo = jnp.maximum(a_ref[0] / cnt, 0.0)
  hi = jnp.maximum(a_ref[1] / cnt, 0.0)
  o_ref[...] = jnp.concatenate([lo, hi], axis=1)


def _phase2(sums, cnts):
  return pl.pallas_call(
      _phase2_body,
      grid=(BATCH // RB,),
      in_specs=[
          pl.BlockSpec((NC, RB, HF), lambda i: (0, i, 0)),
          pl.BlockSpec((NC, RB, CW), lambda i: (0, i, 0)),
      ],
      out_specs=pl.BlockSpec((RB, FEAT), lambda i: (i, 0)),
      out_shape=jax.ShapeDtypeStruct((BATCH, FEAT), jnp.float32),
  )(sums, cnts)


@jax.jit
def kernel(neigh_src_ids, neigh_seg_ids, features_table):
  src = neigh_src_ids.astype(jnp.int32)
  seg3d = neigh_seg_ids.astype(jnp.int32).reshape(NS, NCHUNK, K)
  tbl = features_table.reshape(NUM_NODES * NC, HF)
  sums, cnts = _phase1(src, seg3d, tbl)
  return _phase2(sums, cnts)


# speedup vs baseline: 12.3371x; 1.0078x over previous
import functools

import jax
import jax.numpy as jnp
from jax import lax
from jax.experimental import pallas as pl
from jax.experimental.pallas import tpu as pltpu
from jax.experimental.pallas import tpu_sc as plsc

NUM_NODES = 50000
BATCH = 10000
NUM_EDGES = 320000
FEAT = 128

NC = 2
NS = 16
HF = FEAT // NC
E_TILE = NUM_EDGES // NS
K = 40
NCHUNK = E_TILE // K
BATCHP = 10240
RPT = BATCHP // NS
CW = 16
ZROWS = 64
NBUF = 5
NGRP = NCHUNK // NBUF


def _phase1_body(src_hbm, seg_hbm, tbl_hbm, out_hbm,
                 src_v, seg_v, rows_v, ones_v, zrow_v, zcnt_v,
                 acc_sh, cnt_sh, sem):
  c = lax.axis_index("c")
  s = lax.axis_index("s")

  zero16 = jnp.zeros((16,), jnp.float32)
  one16 = jnp.ones((16,), jnp.float32)

  def init_zrow(i, carry):
    for q in range(HF // 16):
      zrow_v[i, pl.ds(q * 16, 16)] = zero16
    return carry

  lax.fori_loop(0, ZROWS, init_zrow, 0)

  def init_zcnt(i, carry):
    zcnt_v[i, :] = zero16
    return carry

  lax.fori_loop(0, ZROWS, init_zcnt, 0)

  def init_ones(i, carry):
    ones_v[i, :] = one16
    return carry

  lax.fori_loop(0, K, init_ones, 0)

  row0 = s * RPT
  for r in range(RPT // ZROWS):
    pltpu.sync_copy(zrow_v, acc_sh.at[pl.ds(row0 + r * ZROWS, ZROWS)])
  for r in range(RPT // ZROWS):
    pltpu.sync_copy(zcnt_v, cnt_sh.at[pl.ds(row0 + r * ZROWS, ZROWS)])
  plsc.subcore_barrier()

  pltpu.sync_copy(src_hbm.at[pl.ds(s * E_TILE, E_TILE)], src_v)
  pltpu.sync_copy(seg_hbm.at[s], seg_v)

  def mk_idx(i, carry):
    v = src_v[pl.ds(i * 16, 16)]
    src_v[pl.ds(i * 16, 16)] = v * 2 + c
    return carry

  lax.fori_loop(0, E_TILE // 16, mk_idx, 0)

  for b in range(NBUF):
    idx = src_v.at[pl.ds(b * K, K)]
    pltpu.async_copy(tbl_hbm.at[idx], rows_v.at[b], sem.at[b])

  def group(g, carry):
    for b in range(NBUF):
      j = g * NBUF + b
      idx = src_v.at[pl.ds(j * K, K)]
      pltpu.make_async_copy(tbl_hbm.at[idx], rows_v.at[b],
                            sem.at[b]).wait()
      segrow = seg_v.at[j]
      pltpu.sync_copy(rows_v.at[b], acc_sh.at[segrow], add=True)

      pltpu.sync_copy(ones_v, cnt_sh.at[segrow], add=True)

      @pl.when(g < NGRP - 1)
      def _():
        idx2 = src_v.at[pl.ds((j + NBUF) * K, K)]
        pltpu.async_copy(tbl_hbm.at[idx2], rows_v.at[b], sem.at[b])

    return carry

  lax.fori_loop(0, NGRP, group, 0)
  plsc.subcore_barrier()

  def finish_block(b0, nrows):
    pltpu.sync_copy(acc_sh.at[pl.ds(b0, nrows)], zrow_v.at[pl.ds(0, nrows)])
    pltpu.sync_copy(cnt_sh.at[pl.ds(b0, nrows)], zcnt_v.at[pl.ds(0, nrows)])

    def row_fn(r, carry):
      cv = jnp.maximum(zcnt_v[r, :], 1.0)
      for q in range(HF // 16):
        col = pl.ds(q * 16, 16)
        zrow_v[r, col] = jnp.maximum(zrow_v[r, col] / cv, 0.0)
      return carry

    lax.fori_loop(0, nrows, row_fn, 0)
    pltpu.sync_copy(zrow_v.at[pl.ds(0, nrows)],
                    out_hbm.at[pl.ds(b0, nrows), pl.ds(c * HF, HF)])

  @pl.when(s < NS - 1)
  def _():
    for t in range(RPT // ZROWS):
      finish_block(row0 + t * ZROWS, ZROWS)

  @pl.when(s == NS - 1)
  def _():
    last0 = (NS - 1) * RPT
    nfull = (BATCH - last0) // ZROWS
    for t in range(nfull):
      finish_block(last0 + t * ZROWS, ZROWS)
    rem = BATCH - (last0 + nfull * ZROWS)
    finish_block(last0 + nfull * ZROWS, rem)


_phase1 = functools.partial(
    pl.kernel,
    out_type=jax.ShapeDtypeStruct((BATCH, FEAT), jnp.float32),
    mesh=plsc.VectorSubcoreMesh(
        core_axis_name="c", subcore_axis_name="s",
        num_cores=NC, num_subcores=NS),
    scratch_types=[
        pltpu.VMEM((E_TILE,), jnp.int32),
        pltpu.VMEM((NCHUNK, K), jnp.int32),
        pltpu.VMEM((NBUF, K, HF), jnp.float32),
        pltpu.VMEM((K, CW), jnp.float32),
        pltpu.VMEM((ZROWS, HF), jnp.float32),
        pltpu.VMEM((ZROWS, CW), jnp.float32),
        pltpu.VMEM_SHARED((BATCHP, HF), jnp.float32),
        pltpu.VMEM_SHARED((BATCHP, CW), jnp.float32),
        pltpu.SemaphoreType.DMA((NBUF,)),
    ],
    compiler_params=pltpu.CompilerParams(use_tc_tiling_on_sc=False),
)(_phase1_body)


@jax.jit
def kernel(neigh_src_ids, neigh_seg_ids, features_table):
  src = neigh_src_ids.astype(jnp.int32)
  seg3d = neigh_seg_ids.astype(jnp.int32).reshape(NS, NCHUNK, K)
  tbl = features_table.reshape(NUM_NODES * NC, HF)
  return _phase1(src, seg3d, tbl)


# speedup vs baseline: 13.6105x; 1.1032x over previous
import functools

import jax
import jax.numpy as jnp
from jax import lax
from jax.experimental import pallas as pl
from jax.experimental.pallas import tpu as pltpu
from jax.experimental.pallas import tpu_sc as plsc

NUM_NODES = 50000
BATCH = 10000
NUM_EDGES = 320000
FEAT = 128

NC = 2
NS = 16
HF = FEAT // NC
E_TILE = NUM_EDGES // NS
K = 40
NCHUNK = E_TILE // K
BATCHP = 10240
RPT = BATCHP // NS
CW = 16
ZROWS = 64
NBUF = 5
NGRP = NCHUNK // NBUF


def _phase1_body(src_hbm, seg_hbm, tbl_hbm, out_hbm,
                 src_v, seg_v, rows_v, ones_v, zrow_v, zcnt_v,
                 acc_sh, cnt_sh, sem, sem2):
  c = lax.axis_index("c")
  s = lax.axis_index("s")

  zero16 = jnp.zeros((16,), jnp.float32)
  one16 = jnp.ones((16,), jnp.float32)

  def init_zrow(i, carry):
    for q in range(HF // 16):
      zrow_v[i, pl.ds(q * 16, 16)] = zero16
    return carry

  lax.fori_loop(0, ZROWS, init_zrow, 0)

  def init_zcnt(i, carry):
    zcnt_v[i, :] = zero16
    return carry

  lax.fori_loop(0, ZROWS, init_zcnt, 0)

  def init_ones(i, carry):
    ones_v[i, :] = one16
    return carry

  lax.fori_loop(0, K, init_ones, 0)

  row0 = s * RPT
  for r in range(RPT // ZROWS):
    pltpu.sync_copy(zrow_v, acc_sh.at[pl.ds(row0 + r * ZROWS, ZROWS)])
  for r in range(RPT // ZROWS):
    pltpu.sync_copy(zcnt_v, cnt_sh.at[pl.ds(row0 + r * ZROWS, ZROWS)])
  plsc.subcore_barrier()

  pltpu.sync_copy(src_hbm.at[pl.ds(s * E_TILE, E_TILE)], src_v)
  pltpu.sync_copy(seg_hbm.at[s], seg_v)

  def mk_idx(i, carry):
    v = src_v[pl.ds(i * 16, 16)]
    src_v[pl.ds(i * 16, 16)] = v * 2 + c
    return carry

  lax.fori_loop(0, E_TILE // 16, mk_idx, 0)

  for b in range(NBUF):
    idx = src_v.at[pl.ds(b * K, K)]
    pltpu.async_copy(tbl_hbm.at[idx], rows_v.at[b], sem.at[b])

  def wait_scatters(bq, jq):
    segq = seg_v.at[jq]
    pltpu.make_async_copy(rows_v.at[bq], acc_sh.at[segq], sem2.at[bq]).wait()
    pltpu.make_async_copy(ones_v, cnt_sh.at[segq], sem2.at[bq]).wait()

  def group(g, carry):
    for b in range(NBUF):
      j = g * NBUF + b
      idx = src_v.at[pl.ds(j * K, K)]
      pltpu.make_async_copy(tbl_hbm.at[idx], rows_v.at[b],
                            sem.at[b]).wait()
      segrow = seg_v.at[j]
      pltpu.async_copy(rows_v.at[b], acc_sh.at[segrow], sem2.at[b],
                       add=True)
      pltpu.async_copy(ones_v, cnt_sh.at[segrow], sem2.at[b], add=True)

      bp = b - 1 if b > 0 else NBUF - 1
      if b == 0:
        @pl.when(g > 0)
        def _():
          wait_scatters(bp, j - 1)
          idxn = src_v.at[pl.ds((j - 1 + NBUF) * K, K)]
          pltpu.async_copy(tbl_hbm.at[idxn], rows_v.at[bp], sem.at[bp])
      else:
        wait_scatters(bp, j - 1)

        @pl.when(g < NGRP - 1)
        def _():
          idxn = src_v.at[pl.ds((j - 1 + NBUF) * K, K)]
          pltpu.async_copy(tbl_hbm.at[idxn], rows_v.at[bp], sem.at[bp])

    return carry

  lax.fori_loop(0, NGRP, group, 0)
  wait_scatters(NBUF - 1, NCHUNK - 1)
  plsc.subcore_barrier()

  def finish_block(b0, nrows):
    pltpu.sync_copy(acc_sh.at[pl.ds(b0, nrows)], zrow_v.at[pl.ds(0, nrows)])
    pltpu.sync_copy(cnt_sh.at[pl.ds(b0, nrows)], zcnt_v.at[pl.ds(0, nrows)])

    def row_fn(r, carry):
      cv = jnp.maximum(zcnt_v[r, :], 1.0)
      for q in range(HF // 16):
        col = pl.ds(q * 16, 16)
        zrow_v[r, col] = jnp.maximum(zrow_v[r, col] / cv, 0.0)
      return carry

    lax.fori_loop(0, nrows, row_fn, 0)
    pltpu.sync_copy(zrow_v.at[pl.ds(0, nrows)],
                    out_hbm.at[pl.ds(b0, nrows), pl.ds(c * HF, HF)])

  @pl.when(s < NS - 1)
  def _():
    for t in range(RPT // ZROWS):
      finish_block(row0 + t * ZROWS, ZROWS)

  @pl.when(s == NS - 1)
  def _():
    last0 = (NS - 1) * RPT
    nfull = (BATCH - last0) // ZROWS
    for t in range(nfull):
      finish_block(last0 + t * ZROWS, ZROWS)
    rem = BATCH - (last0 + nfull * ZROWS)
    finish_block(last0 + nfull * ZROWS, rem)


_phase1 = functools.partial(
    pl.kernel,
    out_type=jax.ShapeDtypeStruct((BATCH, FEAT), jnp.float32),
    mesh=plsc.VectorSubcoreMesh(
        core_axis_name="c", subcore_axis_name="s",
        num_cores=NC, num_subcores=NS),
    scratch_types=[
        pltpu.VMEM((E_TILE,), jnp.int32),
        pltpu.VMEM((NCHUNK, K), jnp.int32),
        pltpu.VMEM((NBUF, K, HF), jnp.float32),
        pltpu.VMEM((K, CW), jnp.float32),
        pltpu.VMEM((ZROWS, HF), jnp.float32),
        pltpu.VMEM((ZROWS, CW), jnp.float32),
        pltpu.VMEM_SHARED((BATCHP, HF), jnp.float32),
        pltpu.VMEM_SHARED((BATCHP, CW), jnp.float32),
        pltpu.SemaphoreType.DMA((NBUF,)),
        pltpu.SemaphoreType.DMA((NBUF,)),
    ],
    compiler_params=pltpu.CompilerParams(use_tc_tiling_on_sc=False),
)(_phase1_body)


@jax.jit
def kernel(neigh_src_ids, neigh_seg_ids, features_table):
  src = neigh_src_ids.astype(jnp.int32)
  seg3d = neigh_seg_ids.astype(jnp.int32).reshape(NS, NCHUNK, K)
  tbl = features_table.reshape(NUM_NODES * NC, HF)
  return _phase1(src, seg3d, tbl)


# speedup vs baseline: 14.2067x; 1.0438x over previous
import functools

import jax
import jax.numpy as jnp
from jax import lax
from jax.experimental import pallas as pl
from jax.experimental.pallas import tpu as pltpu
from jax.experimental.pallas import tpu_sc as plsc

NUM_NODES = 50000
BATCH = 10000
NUM_EDGES = 320000
FEAT = 128

NC = 2
NS = 16
HF = FEAT // NC
E_TILE = NUM_EDGES // NS
K = 80
NCHUNK = E_TILE // K
BATCHP = 10240
RPT = BATCHP // NS
CW = 16
ZROWS = 64
NBUF = 5
NGRP = NCHUNK // NBUF
SRC_BITS = 17
SRC_MASK = (1 << SRC_BITS) - 1


def _phase1_body(pk_hbm, tbl_hbm, out_hbm,
                 pk_v, idx_s, seg_s, rows_v, ones_v, zrow_v, zcnt_v,
                 acc_sh, cnt_sh, sem, sem2):
  c = lax.axis_index("c")
  s = lax.axis_index("s")

  zero16 = jnp.zeros((16,), jnp.float32)
  one16 = jnp.ones((16,), jnp.float32)

  def init_zrow(i, carry):
    for q in range(HF // 16):
      zrow_v[i, pl.ds(q * 16, 16)] = zero16
    return carry

  lax.fori_loop(0, ZROWS, init_zrow, 0)

  def init_zcnt(i, carry):
    zcnt_v[i, :] = zero16
    return carry

  lax.fori_loop(0, ZROWS, init_zcnt, 0)

  def init_ones(i, carry):
    ones_v[i, :] = one16
    return carry

  lax.fori_loop(0, K, init_ones, 0)

  row0 = s * RPT
  for r in range(RPT // ZROWS):
    pltpu.sync_copy(zrow_v, acc_sh.at[pl.ds(row0 + r * ZROWS, ZROWS)])
  for r in range(RPT // ZROWS):
    pltpu.sync_copy(zcnt_v, cnt_sh.at[pl.ds(row0 + r * ZROWS, ZROWS)])
  plsc.subcore_barrier()

  pltpu.sync_copy(pk_hbm.at[s], pk_v)

  def unpack(chunk, slot):
    for q in range(K // 16):
      col = pl.ds(q * 16, 16)
      p = pk_v[chunk, col]
      idx_s[slot, col] = (p & SRC_MASK) * 2 + c
      seg_s[slot, col] = lax.shift_right_logical(p, SRC_BITS)

  for b in range(NBUF):
    unpack(b, b)
    pltpu.async_copy(tbl_hbm.at[idx_s.at[b]], rows_v.at[b], sem.at[b])

  def wait_scatters(bq):
    pltpu.make_async_copy(rows_v.at[bq], acc_sh.at[seg_s.at[bq]],
                          sem2.at[bq]).wait()
    pltpu.make_async_copy(ones_v, cnt_sh.at[seg_s.at[bq]],
                          sem2.at[bq]).wait()

  def group(g, carry):
    for b in range(NBUF):
      j = g * NBUF + b
      pltpu.make_async_copy(tbl_hbm.at[idx_s.at[b]], rows_v.at[b],
                            sem.at[b]).wait()
      segrow = seg_s.at[b]
      pltpu.async_copy(rows_v.at[b], acc_sh.at[segrow], sem2.at[b],
                       add=True)
      pltpu.async_copy(ones_v, cnt_sh.at[segrow], sem2.at[b], add=True)

      bp = b - 1 if b > 0 else NBUF - 1
      if b == 0:
        @pl.when(g > 0)
        def _():
          wait_scatters(bp)
          unpack(j - 1 + NBUF, bp)
          pltpu.async_copy(tbl_hbm.at[idx_s.at[bp]], rows_v.at[bp],
                           sem.at[bp])
      else:
        wait_scatters(bp)

        @pl.when(g < NGRP - 1)
        def _():
          unpack(j - 1 + NBUF, bp)
          pltpu.async_copy(tbl_hbm.at[idx_s.at[bp]], rows_v.at[bp],
                           sem.at[bp])

    return carry

  lax.fori_loop(0, NGRP, group, 0)
  wait_scatters(NBUF - 1)
  plsc.subcore_barrier()

  def finish_block(b0, nrows):
    pltpu.sync_copy(acc_sh.at[pl.ds(b0, nrows)], zrow_v.at[pl.ds(0, nrows)])
    pltpu.sync_copy(cnt_sh.at[pl.ds(b0, nrows)], zcnt_v.at[pl.ds(0, nrows)])

    def row_fn(r, carry):
      cv = jnp.maximum(zcnt_v[r, :], 1.0)
      for q in range(HF // 16):
        col = pl.ds(q * 16, 16)
        zrow_v[r, col] = jnp.maximum(zrow_v[r, col] / cv, 0.0)
      return carry

    lax.fori_loop(0, nrows, row_fn, 0)
    pltpu.sync_copy(zrow_v.at[pl.ds(0, nrows)],
                    out_hbm.at[pl.ds(b0, nrows), pl.ds(c * HF, HF)])

  @pl.when(s < NS - 1)
  def _():
    for t in range(RPT // ZROWS):
      finish_block(row0 + t * ZROWS, ZROWS)

  @pl.when(s == NS - 1)
  def _():
    last0 = (NS - 1) * RPT
    nfull = (BATCH - last0) // ZROWS
    for t in range(nfull):
      finish_block(last0 + t * ZROWS, ZROWS)
    rem = BATCH - (last0 + nfull * ZROWS)
    finish_block(last0 + nfull * ZROWS, rem)


_phase1 = functools.partial(
    pl.kernel,
    out_type=jax.ShapeDtypeStruct((BATCH, FEAT), jnp.float32),
    mesh=plsc.VectorSubcoreMesh(
        core_axis_name="c", subcore_axis_name="s",
        num_cores=NC, num_subcores=NS),
    scratch_types=[
        pltpu.VMEM((NCHUNK, K), jnp.int32),
        pltpu.VMEM((NBUF, K), jnp.int32),
        pltpu.VMEM((NBUF, K), jnp.int32),
        pltpu.VMEM((NBUF, K, HF), jnp.float32),
        pltpu.VMEM((K, CW), jnp.float32),
        pltpu.VMEM((ZROWS, HF), jnp.float32),
        pltpu.VMEM((ZROWS, CW), jnp.float32),
        pltpu.VMEM_SHARED((BATCHP, HF), jnp.float32),
        pltpu.VMEM_SHARED((BATCHP, CW), jnp.float32),
        pltpu.SemaphoreType.DMA((NBUF,)),
        pltpu.SemaphoreType.DMA((NBUF,)),
    ],
    compiler_params=pltpu.CompilerParams(use_tc_tiling_on_sc=False),
)(_phase1_body)


@jax.jit
def kernel(neigh_src_ids, neigh_seg_ids, features_table):
  src = neigh_src_ids.astype(jnp.int32)
  seg = neigh_seg_ids.astype(jnp.int32)
  packed = jnp.bitwise_or(jnp.left_shift(seg, SRC_BITS), src)
  pk3d = packed.reshape(NS, NCHUNK, K)
  tbl = features_table.reshape(NUM_NODES * NC, HF)
  return _phase1(pk3d, tbl)
